# baseline (speedup 1.0000x reference)
import functools

import jax
import jax.numpy as jnp
from jax import lax
from jax.experimental import pallas as pl
from jax.experimental.pallas import tpu as pltpu
from jax.experimental.pallas import tpu_sc as plsc

B = 16384
D = 64
N = 1000000

NC = 2
NS = 16
NW = NC * NS
BPW = B // NW
CH = BPW // 2


def _sc_gather_body(idx_hbm, tab, out, idx_v, buf0, buf1, sg0, sg1, so0, so1):
    wid = lax.axis_index("s") * NC + lax.axis_index("c")
    base = wid * BPW
    pltpu.sync_copy(idx_hbm.at[pl.ds(base, BPW)], idx_v)
    g0 = pltpu.async_copy(tab.at[idx_v.at[pl.ds(0, CH)]], buf0, sg0)
    g1 = pltpu.async_copy(tab.at[idx_v.at[pl.ds(CH, CH)]], buf1, sg1)
    g0.wait()
    o0 = pltpu.async_copy(buf0, out.at[pl.ds(base, CH)], so0)
    g1.wait()
    o1 = pltpu.async_copy(buf1, out.at[pl.ds(base + CH, CH)], so1)
    o0.wait()
    o1.wait()


@functools.cache
def _sc_gather():
    mesh = plsc.VectorSubcoreMesh(
        core_axis_name="c", subcore_axis_name="s", num_cores=NC, num_subcores=NS
    )
    return pl.kernel(
        _sc_gather_body,
        out_type=jax.ShapeDtypeStruct((B, 128), jnp.int32),
        mesh=mesh,
        scratch_types=[
            pltpu.VMEM((BPW,), jnp.int32),
            pltpu.VMEM((CH, 128), jnp.int32),
            pltpu.VMEM((CH, 128), jnp.int32),
            pltpu.SemaphoreType.DMA,
            pltpu.SemaphoreType.DMA,
            pltpu.SemaphoreType.DMA,
            pltpu.SemaphoreType.DMA,
        ],
    )


def _unpack(w_ref, p_ref):
    w = w_ref[...]
    p = p_ref[...] > 0
    half = jnp.where(p, w[:, 64:], w[:, :64])
    lo = lax.bitcast_convert_type(half << 16, jnp.float32)
    hi = lax.bitcast_convert_type(half & jnp.int32(-65536), jnp.float32)
    return jnp.concatenate([lo, hi], axis=1)


def _tc_body(su_ref, si_ref, pu_ref, pi_ref,
             w1_ref, b1_ref, w2_ref, b2_ref, w3_ref, b3_ref,
             wp_ref, bp_ref, out_ref):
    su = _unpack(su_ref, pu_ref)
    si = _unpack(si_ref, pi_ref)
    gu = jnp.concatenate([su[:, :32], su[:, 64:96]], axis=1)
    mu = jnp.concatenate([su[:, 32:64], su[:, 96:128]], axis=1)
    gi = jnp.concatenate([si[:, :32], si[:, 64:96]], axis=1)
    mi = jnp.concatenate([si[:, 32:64], si[:, 96:128]], axis=1)
    w1 = w1_ref[...]
    h = jnp.dot(mu, w1[:D], preferred_element_type=jnp.float32)
    h = h + jnp.dot(mi, w1[D:], preferred_element_type=jnp.float32)
    h = jnp.maximum(h + b1_ref[...], 0.0)
    h = jnp.maximum(
        jnp.dot(h, w2_ref[...], preferred_element_type=jnp.float32) + b2_ref[...], 0.0)
    h = jnp.maximum(
        jnp.dot(h, w3_ref[...], preferred_element_type=jnp.float32) + b3_ref[...], 0.0)
    g = gu * gi
    wp = wp_ref[...]
    pred = jnp.dot(g, wp[:D], preferred_element_type=jnp.float32)
    pred = pred + jnp.dot(h, wp[D:], preferred_element_type=jnp.float32)
    out_ref[...] = pred + bp_ref[...]


def _tc_dense(su, si, pu, pi_, W1, b1, W2, b2, W3, b3, Wp, bp):
    R = 2048
    grid = (B // R,)
    row_spec = pl.BlockSpec((R, 128), lambda r: (r, 0))
    par_spec = pl.BlockSpec((R, 1), lambda r: (r, 0))

    def full(shape):
        return pl.BlockSpec(shape, lambda r: (0,) * len(shape))

    return pl.pallas_call(
        _tc_body,
        grid=grid,
        in_specs=[
            row_spec, row_spec, par_spec, par_spec,
            full(W1.shape), full((1, b1.shape[0])),
            full(W2.shape), full((1, b2.shape[0])),
            full(W3.shape), full((1, b3.shape[0])),
            full(Wp.shape), full((1, 1)),
        ],
        out_specs=pl.BlockSpec((R, 1), lambda r: (r, 0)),
        out_shape=jax.ShapeDtypeStruct((B, 1), jnp.float32),
    )(su, si, pu, pi_, W1, b1.reshape(1, -1), W2, b2.reshape(1, -1),
      W3, b3.reshape(1, -1), Wp, bp.reshape(1, 1))


def _b16(x):
    u = lax.bitcast_convert_type(x, jnp.uint32)
    return (u + jnp.uint32(0x7FFF) + ((u >> 16) & jnp.uint32(1))) >> 16


def _pack(a, b):
    t = jnp.concatenate([_b16(a), _b16(b)], axis=1)
    w = (t[:, 1::2] << 16) | t[:, 0::2]
    return lax.bitcast_convert_type(w, jnp.int32).reshape(N // 2, 128)


def kernel(u, i, gmf_user_table, gmf_item_table, mlp_user_table, mlp_item_table,
           W1, b1, W2, b2, W3, b3, Wp, bp):
    u = u.astype(jnp.int32)
    i = i.astype(jnp.int32)
    gather = _sc_gather()
    tu = _pack(gmf_user_table, mlp_user_table)
    ti = _pack(gmf_item_table, mlp_item_table)
    su = gather(u >> 1, tu)
    si = gather(i >> 1, ti)
    pu = (u & 1).reshape(B, 1)
    pi_ = (i & 1).reshape(B, 1)
    eo = jnp.concatenate([jnp.arange(0, D, 2), jnp.arange(1, D, 2)])
    W1p = jnp.concatenate([W1[:D][eo], W1[D:][eo]], axis=0)
    Wpp = jnp.concatenate([Wp[:D][eo], Wp[D:]], axis=0)
    out = _tc_dense(su, si, pu, pi_, W1p, b1, W2, b2, W3, b3, Wpp, bp)
    return out[:, 0]


# speedup vs baseline: 4.5317x; 4.5317x over previous
import functools

import jax
import jax.numpy as jnp
from jax import lax
from jax.experimental import pallas as pl
from jax.experimental.pallas import tpu as pltpu
from jax.experimental.pallas import tpu_sc as plsc

B = 16384
D = 64
N = 1000000

NC = 2
NS = 16
NW = NC * NS
BPW = B // NW
CH = BPW // 2


def _sc_gather_body(idx_hbm, tab, out, idx_v, buf0, buf1, sg0, sg1, so0, so1):
    wid = lax.axis_index("s") * NC + lax.axis_index("c")
    base = wid * BPW
    pltpu.sync_copy(idx_hbm.at[pl.ds(base, BPW)], idx_v)
    g0 = pltpu.async_copy(tab.at[idx_v.at[pl.ds(0, CH)]], buf0, sg0)
    g1 = pltpu.async_copy(tab.at[idx_v.at[pl.ds(CH, CH)]], buf1, sg1)
    g0.wait()
    o0 = pltpu.async_copy(buf0, out.at[pl.ds(base, CH)], so0)
    g1.wait()
    o1 = pltpu.async_copy(buf1, out.at[pl.ds(base + CH, CH)], so1)
    o0.wait()
    o1.wait()


@functools.cache
def _sc_gather():
    mesh = plsc.VectorSubcoreMesh(
        core_axis_name="c", subcore_axis_name="s", num_cores=NC, num_subcores=NS
    )
    return pl.kernel(
        _sc_gather_body,
        out_type=jax.ShapeDtypeStruct((B, 128), jnp.float32),
        mesh=mesh,
        scratch_types=[
            pltpu.VMEM((BPW,), jnp.int32),
            pltpu.VMEM((CH, 128), jnp.float32),
            pltpu.VMEM((CH, 128), jnp.float32),
            pltpu.SemaphoreType.DMA,
            pltpu.SemaphoreType.DMA,
            pltpu.SemaphoreType.DMA,
            pltpu.SemaphoreType.DMA,
        ],
    )


def _tc_body(su_ref, si_ref,
             w1_ref, b1_ref, w2_ref, b2_ref, w3_ref, b3_ref,
             wp_ref, bp_ref, out_ref):
    su = su_ref[...].astype(jnp.float32)
    si = si_ref[...].astype(jnp.float32)
    gu, mu = su[:, :D], su[:, D:]
    gi, mi = si[:, :D], si[:, D:]
    w1 = w1_ref[...]
    h = jnp.dot(mu, w1[:D], preferred_element_type=jnp.float32)
    h = h + jnp.dot(mi, w1[D:], preferred_element_type=jnp.float32)
    h = jnp.maximum(h + b1_ref[...], 0.0)
    h = jnp.maximum(
        jnp.dot(h, w2_ref[...], preferred_element_type=jnp.float32) + b2_ref[...], 0.0)
    h = jnp.maximum(
        jnp.dot(h, w3_ref[...], preferred_element_type=jnp.float32) + b3_ref[...], 0.0)
    g = gu * gi
    wp = wp_ref[...]
    pred = jnp.dot(g, wp[:D], preferred_element_type=jnp.float32)
    pred = pred + jnp.dot(h, wp[D:], preferred_element_type=jnp.float32)
    out_ref[...] = pred + bp_ref[...]


def _tc_dense(su, si, W1, b1, W2, b2, W3, b3, Wp, bp):
    R = 2048
    grid = (B // R,)
    row_spec = pl.BlockSpec((R, 128), lambda r: (r, 0))

    def full(shape):
        return pl.BlockSpec(shape, lambda r: (0,) * len(shape))

    return pl.pallas_call(
        _tc_body,
        grid=grid,
        in_specs=[
            row_spec, row_spec,
            full(W1.shape), full((1, b1.shape[0])),
            full(W2.shape), full((1, b2.shape[0])),
            full(W3.shape), full((1, b3.shape[0])),
            full(Wp.shape), full((1, 1)),
        ],
        out_specs=pl.BlockSpec((R, 1), lambda r: (r, 0)),
        out_shape=jax.ShapeDtypeStruct((B, 1), jnp.float32),
    )(su, si, W1, b1.reshape(1, -1), W2, b2.reshape(1, -1),
      W3, b3.reshape(1, -1), Wp, bp.reshape(1, 1))


def kernel(u, i, gmf_user_table, gmf_item_table, mlp_user_table, mlp_item_table,
           W1, b1, W2, b2, W3, b3, Wp, bp):
    u = u.astype(jnp.int32)
    i = i.astype(jnp.int32)
    gather = _sc_gather()
    tu = jnp.concatenate([gmf_user_table, mlp_user_table], axis=1)
    ti = jnp.concatenate([gmf_item_table, mlp_item_table], axis=1)
    su = gather(u, tu)
    si = gather(i, ti)
    out = _tc_dense(su, si, W1, b1, W2, b2, W3, b3, Wp, bp)
    return out[:, 0]
